# baseline (speedup 1.0000x reference)
import jax
import jax.numpy as jnp
from jax import lax
from jax.experimental import pallas as pl
from jax.experimental.pallas import tpu as pltpu
from jax.experimental.pallas import tpu_sc as plsc

B = 16384
CTX = 20
TGT = 50
D = 64
L = 16

_info = plsc.get_sparse_core_info()
NC = _info.num_cores
NS = _info.num_subcores
NW = NC * NS

ROWS_PER_W = B // NW
R = 4
NCHUNK = ROWS_PER_W // R
CTX_COLS = 80
TGT_COLS = 100
CTX_IDX_ROWS = ROWS_PER_W * CTX // CTX_COLS
TGT_IDX_ROWS = ROWS_PER_W * TGT // TGT_COLS


def _body(w_hbm, ctx_hbm, tgt_hbm, out_hbm,
          ctx_idx, tgt_idx, ctx_rows0, tgt_rows0, ctx_rows1, tgt_rows1,
          scores, sem0, sem1):
    wid = lax.axis_index("s") * NC + lax.axis_index("c")
    pltpu.sync_copy(ctx_hbm.at[pl.ds(wid * CTX_IDX_ROWS, CTX_IDX_ROWS)], ctx_idx)
    pltpu.sync_copy(tgt_hbm.at[pl.ds(wid * TGT_IDX_ROWS, TGT_IDX_ROWS)], tgt_idx)
    lanes = lax.iota(jnp.int32, L)
    zero = jnp.zeros((L,), jnp.float32)
    perms = [lanes ^ sh for sh in (8, 4, 2, 1)]

    def copies(c, ctx_buf, tgt_buf, sem):
        cps = [pltpu.make_async_copy(
            w_hbm.at[ctx_idx.at[c]], ctx_buf, sem)]
        for j in range(2):
            cps.append(pltpu.make_async_copy(
                w_hbm.at[tgt_idx.at[2 * c + j]],
                tgt_buf.at[pl.ds(j * TGT_COLS, TGT_COLS)], sem))
        return cps

    def issue(c, ctx_buf, tgt_buf, sem):
        for cp in copies(c, ctx_buf, tgt_buf, sem):
            cp.start()

    def drain(c, ctx_buf, tgt_buf, sem):
        for cp in copies(c, ctx_buf, tgt_buf, sem):
            cp.wait()

    def compute(c, ctx_buf, tgt_buf):
        base = wid * ROWS_PER_W + c * R
        for r in range(R):
            def vc_body(t, acc):
                row = r * CTX + t
                return tuple(acc[k] + ctx_buf[row, pl.ds(L * k, L)]
                             for k in range(4))
            acc = lax.fori_loop(0, CTX, vc_body, (zero, zero, zero, zero),
                                unroll=4)
            vc = tuple(acc[k] / float(CTX) for k in range(4))

            for tg in range(4):
                rbase = r * TGT + tg * L

                def t_body(t, score):
                    row = rbase + t
                    acc = tgt_buf[row, pl.ds(0, L)] * vc[0]
                    for k in range(1, 4):
                        acc = acc + tgt_buf[row, pl.ds(L * k, L)] * vc[k]
                    for p in perms:
                        acc = acc + acc.at[p].get(mode="promise_in_bounds")
                    return jnp.where(lanes == t, acc, score)

                score = lax.fori_loop(0, L, t_body, zero, unroll=4)
                scores[r, pl.ds(tg * L, L)] = score
        pltpu.sync_copy(scores, out_hbm.at[pl.ds(base, R)])

    issue(0, ctx_rows0, tgt_rows0, sem0)

    def pair_body(i, carry):
        c0 = 2 * i
        issue(c0 + 1, ctx_rows1, tgt_rows1, sem1)
        drain(c0, ctx_rows0, tgt_rows0, sem0)
        compute(c0, ctx_rows0, tgt_rows0)

        @pl.when(i < NCHUNK // 2 - 1)
        def _():
            issue(c0 + 2, ctx_rows0, tgt_rows0, sem0)

        drain(c0 + 1, ctx_rows1, tgt_rows1, sem1)
        compute(c0 + 1, ctx_rows1, tgt_rows1)
        return carry

    lax.fori_loop(0, NCHUNK // 2, pair_body, 0)


def kernel(context, targets, W):
    wp = jnp.pad(W, ((0, 0), (0, 128 - D)))
    wv = wp.reshape(2 * W.shape[0], D)
    ctx2d = (context * 2).reshape(B * CTX // CTX_COLS, CTX_COLS)
    tgt2d = (targets * 2).reshape(B * TGT // TGT_COLS, TGT_COLS)
    mesh = plsc.VectorSubcoreMesh(core_axis_name="c", subcore_axis_name="s")
    run = pl.kernel(
        _body,
        out_type=jax.ShapeDtypeStruct((B, D), jnp.float32),
        mesh=mesh,
        compiler_params=pltpu.CompilerParams(use_tc_tiling_on_sc=False),
        scratch_types=[
            pltpu.VMEM((CTX_IDX_ROWS, CTX_COLS), jnp.int32),
            pltpu.VMEM((TGT_IDX_ROWS, TGT_COLS), jnp.int32),
            pltpu.VMEM((CTX_COLS, D), jnp.float32),
            pltpu.VMEM((2 * TGT_COLS + L, D), jnp.float32),
            pltpu.VMEM((CTX_COLS, D), jnp.float32),
            pltpu.VMEM((2 * TGT_COLS + L, D), jnp.float32),
            pltpu.VMEM((R, D), jnp.float32),
            pltpu.SemaphoreType.DMA,
            pltpu.SemaphoreType.DMA,
        ],
    )
    out = run(wv, ctx2d, tgt2d)
    return out[:, :TGT]


# speedup vs baseline: 1.1026x; 1.1026x over previous
import jax
import jax.numpy as jnp
from jax import lax
from jax.experimental import pallas as pl
from jax.experimental.pallas import tpu as pltpu
from jax.experimental.pallas import tpu_sc as plsc

B = 16384
CTX = 20
TGT = 50
D = 64
L = 16

_info = plsc.get_sparse_core_info()
NC = _info.num_cores
NS = _info.num_subcores
NW = NC * NS

ROWS_PER_W = B // NW
R = 8
NCHUNK = ROWS_PER_W // R
CTX_COLS = 80
TGT_COLS = 100
CTX_IDX_ROWS = ROWS_PER_W * CTX // CTX_COLS
TGT_IDX_ROWS = ROWS_PER_W * TGT // TGT_COLS


def _body(w_hbm, ctx_hbm, tgt_hbm, out_hbm,
          ctx_idx, tgt_idx, ctx_rows0, tgt_rows0, ctx_rows1, tgt_rows1,
          scores, sem0, sem1):
    wid = lax.axis_index("s") * NC + lax.axis_index("c")
    pltpu.sync_copy(ctx_hbm.at[pl.ds(wid * CTX_IDX_ROWS, CTX_IDX_ROWS)], ctx_idx)
    pltpu.sync_copy(tgt_hbm.at[pl.ds(wid * TGT_IDX_ROWS, TGT_IDX_ROWS)], tgt_idx)
    lanes = lax.iota(jnp.int32, L)
    zero = jnp.zeros((L,), jnp.float32)
    levels = [(sh, lanes ^ sh, (lanes & sh) == 0) for sh in (8, 4, 2, 1)]
    bitrev = (((lanes & 1) << 3) | ((lanes & 2) << 1)
              | ((lanes & 4) >> 1) | ((lanes & 8) >> 3))

    def copies(c, ctx_buf, tgt_buf, sem):
        cps = []
        for j in range(2):
            cps.append(pltpu.make_async_copy(
                w_hbm.at[ctx_idx.at[2 * c + j]],
                ctx_buf.at[pl.ds(j * CTX_COLS, CTX_COLS)], sem))
        for j in range(4):
            cps.append(pltpu.make_async_copy(
                w_hbm.at[tgt_idx.at[4 * c + j]],
                tgt_buf.at[pl.ds(j * TGT_COLS, TGT_COLS)], sem))
        return cps

    def issue(c, ctx_buf, tgt_buf, sem):
        for cp in copies(c, ctx_buf, tgt_buf, sem):
            cp.start()

    def drain(c, ctx_buf, tgt_buf, sem):
        for cp in copies(c, ctx_buf, tgt_buf, sem):
            cp.wait()

    def compute(c, ctx_buf, tgt_buf):
        base = wid * ROWS_PER_W + c * R
        for r in range(R):
            def vc_body(t, acc):
                row = r * CTX + t
                return tuple(acc[k] + ctx_buf[row, pl.ds(L * k, L)]
                             for k in range(4))
            acc = lax.fori_loop(0, CTX, vc_body, (zero, zero, zero, zero),
                                unroll=4)
            vc = tuple(acc[k] / float(CTX) for k in range(4))

            def tg_body(tg, carry):
                rbase = r * TGT + tg * L
                vs = []
                for t in range(L):
                    row = rbase + t
                    a = tgt_buf[row, pl.ds(0, L)] * vc[0]
                    for k in range(1, 4):
                        a = a + tgt_buf[row, pl.ds(L * k, L)] * vc[k]
                    vs.append(a)
                for s, pm, m in levels:
                    vs = [jnp.where(
                        m,
                        vs[2 * i] + vs[2 * i].at[pm].get(
                            mode="promise_in_bounds"),
                        vs[2 * i + 1] + vs[2 * i + 1].at[pm].get(
                            mode="promise_in_bounds"))
                        for i in range(len(vs) // 2)]
                out = vs[0].at[bitrev].get(mode="promise_in_bounds")
                scores[r, pl.ds(tg * L, L)] = out
                return carry

            lax.fori_loop(0, 4, tg_body, 0)
        pltpu.sync_copy(scores, out_hbm.at[pl.ds(base, R)])

    issue(0, ctx_rows0, tgt_rows0, sem0)

    def pair_body(i, carry):
        c0 = 2 * i
        issue(c0 + 1, ctx_rows1, tgt_rows1, sem1)
        drain(c0, ctx_rows0, tgt_rows0, sem0)
        compute(c0, ctx_rows0, tgt_rows0)

        @pl.when(i < NCHUNK // 2 - 1)
        def _():
            issue(c0 + 2, ctx_rows0, tgt_rows0, sem0)

        drain(c0 + 1, ctx_rows1, tgt_rows1, sem1)
        compute(c0 + 1, ctx_rows1, tgt_rows1)
        return carry

    lax.fori_loop(0, NCHUNK // 2, pair_body, 0)


def kernel(context, targets, W):
    wp = jnp.pad(W, ((0, 0), (0, 128 - D)))
    wv = wp.reshape(2 * W.shape[0], D)
    ctx2d = (context * 2).reshape(B * CTX // CTX_COLS, CTX_COLS)
    tgt2d = (targets * 2).reshape(B * TGT // TGT_COLS, TGT_COLS)
    mesh = plsc.VectorSubcoreMesh(core_axis_name="c", subcore_axis_name="s")
    run = pl.kernel(
        _body,
        out_type=jax.ShapeDtypeStruct((B, D), jnp.float32),
        mesh=mesh,
        compiler_params=pltpu.CompilerParams(use_tc_tiling_on_sc=False),
        scratch_types=[
            pltpu.VMEM((CTX_IDX_ROWS, CTX_COLS), jnp.int32),
            pltpu.VMEM((TGT_IDX_ROWS, TGT_COLS), jnp.int32),
            pltpu.VMEM((2 * CTX_COLS, D), jnp.float32),
            pltpu.VMEM((4 * TGT_COLS + L, D), jnp.float32),
            pltpu.VMEM((2 * CTX_COLS, D), jnp.float32),
            pltpu.VMEM((4 * TGT_COLS + L, D), jnp.float32),
            pltpu.VMEM((R, D), jnp.float32),
            pltpu.SemaphoreType.DMA,
            pltpu.SemaphoreType.DMA,
        ],
    )
    out = run(wv, ctx2d, tgt2d)
    return out[:, :TGT]
